# Initial kernel scaffold; baseline (speedup 1.0000x reference)
import jax
import jax.numpy as jnp
from jax.experimental import pallas as pl


def kernel(x, w1, b1, w2, b2):
    raise NotImplementedError("write your pallas kernel here")


# speedup vs baseline: 1.3347x; 1.3347x over previous
import functools

import jax
import jax.numpy as jnp
from jax.experimental import pallas as pl
from jax.experimental.pallas import tpu as pltpu


def _se_fused_kernel(x_ref, w1t_ref, b1_ref, w2t_ref, b2_ref, o_ref, *, inv_hw):
    x = x_ref[...]
    pooled = jnp.sum(x, axis=2) * inv_hw
    h = jnp.dot(pooled, w1t_ref[...],
                preferred_element_type=jnp.float32) + b1_ref[...]
    h = jnp.maximum(h, 0.0)
    z = jnp.dot(h, w2t_ref[...],
                preferred_element_type=jnp.float32) + b2_ref[...]
    gate = jnp.clip((z + 3.0) * (1.0 / 6.0), 0.0, 1.0)
    o_ref[...] = x * gate[:, :, None]


def kernel(x, w1, b1, w2, b2):
    N, C, H, W = x.shape
    C_r = w1.shape[0]
    HW = H * W
    dtype = x.dtype

    B = 8
    while N % B:
        B //= 2

    xf = x.reshape(N, C, HW)
    w1t = jnp.asarray(w1, jnp.float32).T.reshape(C, C_r)
    b1r = jnp.asarray(b1, jnp.float32).reshape(1, C_r)
    w2t = jnp.asarray(w2, jnp.float32).T.reshape(C_r, C)
    b2r = jnp.asarray(b2, jnp.float32).reshape(1, C)

    out = pl.pallas_call(
        functools.partial(_se_fused_kernel, inv_hw=1.0 / float(HW)),
        out_shape=jax.ShapeDtypeStruct((N, C, HW), dtype),
        grid=(N // B,),
        in_specs=[
            pl.BlockSpec((B, C, HW), lambda n: (n, 0, 0)),
            pl.BlockSpec((C, C_r), lambda n: (0, 0)),
            pl.BlockSpec((1, C_r), lambda n: (0, 0)),
            pl.BlockSpec((C_r, C), lambda n: (0, 0)),
            pl.BlockSpec((1, C), lambda n: (0, 0)),
        ],
        out_specs=pl.BlockSpec((B, C, HW), lambda n: (n, 0, 0)),
        compiler_params=pltpu.CompilerParams(
            dimension_semantics=("parallel",),
            vmem_limit_bytes=100 << 20),
    )(xf, w1t, b1r, w2t, b2r)
    return out.reshape(N, C, H, W)


# speedup vs baseline: 4.9117x; 3.6800x over previous
import functools

import jax
import jax.numpy as jnp
from jax.experimental import pallas as pl
from jax.experimental.pallas import tpu as pltpu


def _se_fused_kernel(x_ref, w1_ref, b1_ref, w2_ref, b2_ref, o_ref, *, inv_hw):
    x = x_ref[...]
    pooled = jnp.sum(x, axis=1) * inv_hw
    h = jax.lax.dot_general(pooled, w1_ref[...],
                            (((1,), (1,)), ((), ())),
                            preferred_element_type=jnp.float32)
    h = jnp.maximum(h + b1_ref[...], 0.0)
    z = jax.lax.dot_general(h, w2_ref[...],
                            (((1,), (1,)), ((), ())),
                            preferred_element_type=jnp.float32)
    z = z + b2_ref[...]
    gate = jnp.clip((z + 3.0) * (1.0 / 6.0), 0.0, 1.0)
    o_ref[...] = x * gate[:, None, :]


def kernel(x, w1, b1, w2, b2):
    N, C, H, W = x.shape
    C_r = w1.shape[0]
    HW = H * W
    dtype = x.dtype

    B = 8
    while N % B:
        B //= 2

    xt = jnp.transpose(x, (0, 2, 3, 1)).reshape(N, HW, C)

    out = pl.pallas_call(
        functools.partial(_se_fused_kernel, inv_hw=1.0 / float(HW)),
        out_shape=jax.ShapeDtypeStruct((N, HW, C), dtype),
        grid=(N // B,),
        in_specs=[
            pl.BlockSpec((B, HW, C), lambda n: (n, 0, 0)),
            pl.BlockSpec((C_r, C), lambda n: (0, 0)),
            pl.BlockSpec((1, C_r), lambda n: (0, 0)),
            pl.BlockSpec((C, C_r), lambda n: (0, 0)),
            pl.BlockSpec((1, C), lambda n: (0, 0)),
        ],
        out_specs=pl.BlockSpec((B, HW, C), lambda n: (n, 0, 0)),
        compiler_params=pltpu.CompilerParams(
            dimension_semantics=("parallel",),
            vmem_limit_bytes=100 << 20),
    )(xt, w1, b1.reshape(1, C_r), w2, b2.reshape(1, C))

    return out.reshape(N, H, W, C).transpose(0, 3, 1, 2)


# speedup vs baseline: 5.3617x; 1.0916x over previous
import functools

import jax
import jax.numpy as jnp
from jax.experimental import pallas as pl
from jax.experimental.pallas import tpu as pltpu


def _se_fused_kernel(x_ref, w1_ref, b1_ref, w2_ref, b2_ref, o_ref, *, inv_hw):
    x = x_ref[...]
    pooled = jnp.sum(x, axis=1) * inv_hw
    h = jax.lax.dot_general(pooled, w1_ref[...],
                            (((1,), (1,)), ((), ())),
                            preferred_element_type=jnp.float32)
    h = jnp.maximum(h + b1_ref[...], 0.0)
    z = jax.lax.dot_general(h, w2_ref[...],
                            (((1,), (1,)), ((), ())),
                            preferred_element_type=jnp.float32)
    z = z + b2_ref[...]
    gate = jnp.clip((z + 3.0) * (1.0 / 6.0), 0.0, 1.0)
    o_ref[...] = x * gate[:, None, :]


def kernel(x, w1, b1, w2, b2):
    N, C, H, W = x.shape
    C_r = w1.shape[0]
    HW = H * W
    dtype = x.dtype

    B = 16
    while N % B:
        B //= 2

    xt = jnp.transpose(x, (0, 2, 3, 1)).reshape(N, HW, C)

    out = pl.pallas_call(
        functools.partial(_se_fused_kernel, inv_hw=1.0 / float(HW)),
        out_shape=jax.ShapeDtypeStruct((N, HW, C), dtype),
        grid=(N // B,),
        in_specs=[
            pl.BlockSpec((B, HW, C), lambda n: (n, 0, 0)),
            pl.BlockSpec((C_r, C), lambda n: (0, 0)),
            pl.BlockSpec((1, C_r), lambda n: (0, 0)),
            pl.BlockSpec((C, C_r), lambda n: (0, 0)),
            pl.BlockSpec((1, C), lambda n: (0, 0)),
        ],
        out_specs=pl.BlockSpec((B, HW, C), lambda n: (n, 0, 0)),
        compiler_params=pltpu.CompilerParams(
            dimension_semantics=("parallel",),
            vmem_limit_bytes=100 << 20),
    )(xt, w1, b1.reshape(1, C_r), w2, b2.reshape(1, C))

    return out.reshape(N, H, W, C).transpose(0, 3, 1, 2)
